# Initial kernel scaffold; baseline (speedup 1.0000x reference)
import jax
import jax.numpy as jnp
from jax.experimental import pallas as pl


def kernel(uList, table):
    raise NotImplementedError("write your pallas kernel here")


# speedup vs baseline: 1.3075x; 1.3075x over previous
import functools

import jax
import jax.numpy as jnp
from jax import lax
from jax.experimental import pallas as pl
from jax.experimental.pallas import tpu as pltpu
from jax.experimental.pallas import tpu_sc as plsc

_RES = 1000000
_LAT = 64
_N = 1048576
_NC = 2
_NS = 16
_NW = _NC * _NS
_BW = _N // _NW
_C = 512
_G = _BW // _C
_SUB = _C // 128


def _body(u_hbm, table_hbm, out_hbm, u_v, idx_a, idx_b, rows_a, rows_b, sem):
    wid = lax.axis_index("s") * _NC + lax.axis_index("c")
    scale = jnp.float32(_RES - 1)

    def chunk(g, carry):
        base = wid * _BW + g * _C
        pltpu.sync_copy(u_hbm.at[pl.ds(base, _C)], u_v)

        def idx_body(j, c):
            for k in range(8):
                off = j * 128 + k * 16
                u16 = u_v[pl.ds(off, 16)]
                f = u16 * scale
                ix = f.astype(jnp.int32)
                fl = ix.astype(jnp.float32)
                idx_a[j, pl.ds(k * 16, 16)] = ix
                idx_b[j, pl.ds(k * 16, 16)] = ix + 1
                u_v[pl.ds(off, 16)] = f - fl
            return c

        lax.fori_loop(0, _SUB, idx_body, 0, unroll=False)

        copies = []
        for j in range(_SUB):
            copies.append(pltpu.async_copy(
                table_hbm.at[idx_a.at[j]], rows_a.at[pl.ds(j * 128, 128)], sem))
            copies.append(pltpu.async_copy(
                table_hbm.at[idx_b.at[j]], rows_b.at[pl.ds(j * 128, 128)], sem))
        for c in copies:
            c.wait()

        def lerp_body(blk, c):
            i0 = blk * 16
            av = u_v[pl.ds(i0, 16)]
            for l in range(16):
                al = jnp.full((16,), av[l], jnp.float32)
                for r in range(4):
                    a = rows_a[i0 + l, pl.ds(r * 16, 16)]
                    b = rows_b[i0 + l, pl.ds(r * 16, 16)]
                    rows_a[i0 + l, pl.ds(r * 16, 16)] = a + al * (b - a)
            return c

        lax.fori_loop(0, _C // 16, lerp_body, 0, unroll=False)

        pltpu.sync_copy(rows_a, out_hbm.at[pl.ds(base, _C)])
        return carry

    lax.fori_loop(0, _G, chunk, 0, unroll=False)


def kernel(uList, table):
    mesh = plsc.VectorSubcoreMesh(core_axis_name="c", subcore_axis_name="s")
    k = functools.partial(
        pl.kernel,
        mesh=mesh,
        out_type=jax.ShapeDtypeStruct((_N, _LAT), jnp.float32),
        compiler_params=pltpu.CompilerParams(use_tc_tiling_on_sc=False),
        scratch_types=[
            pltpu.VMEM((_C,), jnp.float32),
            pltpu.VMEM((_SUB, 128), jnp.int32),
            pltpu.VMEM((_SUB, 128), jnp.int32),
            pltpu.VMEM((_C, _LAT), jnp.float32),
            pltpu.VMEM((_C, _LAT), jnp.float32),
            pltpu.SemaphoreType.DMA,
        ],
    )(_body)
    return k(uList, table)
